# Initial kernel scaffold; baseline (speedup 1.0000x reference)
import jax
import jax.numpy as jnp
from jax.experimental import pallas as pl


def kernel(x, pos_seq, position_weight):
    raise NotImplementedError("write your pallas kernel here")


# speedup vs baseline: 1.7800x; 1.7800x over previous
import functools

import jax
import jax.numpy as jnp
from jax import lax
from jax.experimental import pallas as pl
from jax.experimental.pallas import tpu as pltpu
from jax.experimental.pallas import tpu_sc as plsc

BATCH = 1024
SEQ = 200
D = 64
N = BATCH * SEQ
NC, NS = 2, 16
NW = NC * NS
R = N // NW
C = 128
G = R // C

_mesh = plsc.VectorSubcoreMesh(core_axis_name="c", subcore_axis_name="s")


@functools.partial(
    pl.kernel,
    mesh=_mesh,
    compiler_params=pltpu.CompilerParams(use_tc_tiling_on_sc=False),
    out_type=jax.ShapeDtypeStruct((N, D), jnp.float32),
    scratch_types=[
        pltpu.VMEM((R,), jnp.int32),
        pltpu.VMEM((C, D), jnp.float32),
        pltpu.VMEM((C, D), jnp.float32),
        pltpu.SemaphoreType.DMA,
        pltpu.SemaphoreType.DMA,
    ],
)
def _emb_add(x_hbm, idx_hbm, w_hbm, out_hbm, idx_v, x_v, w_v, sem_x, sem_w):
    wid = lax.axis_index("s") * NC + lax.axis_index("c")
    base = wid * R
    pltpu.sync_copy(idx_hbm.at[pl.ds(base, R)], idx_v)

    def chunk(g, carry):
        r0 = base + g * C
        cpx = pltpu.make_async_copy(x_hbm.at[pl.ds(r0, C)], x_v, sem_x)
        cpw = pltpu.make_async_copy(w_hbm.at[idx_v.at[pl.ds(g * C, C)]], w_v, sem_w)
        cpx.start()
        cpw.start()
        cpx.wait()
        cpw.wait()

        def row(i, c2):
            for c in range(D // 16):
                sl = pl.ds(c * 16, 16)
                plsc.addupdate(x_v.at[i, sl], w_v[i, sl])
            return c2

        lax.fori_loop(0, C, row, 0)
        pltpu.sync_copy(x_v, out_hbm.at[pl.ds(r0, C)])
        return carry

    lax.fori_loop(0, G, chunk, 0)


def kernel(x, pos_seq, position_weight):
    x2 = x.reshape(N, D)
    idx = pos_seq.reshape(N)
    out = _emb_add(x2, idx, position_weight)
    return out.reshape(BATCH, SEQ, D)
